# baseline (speedup 1.0000x reference)
import functools

import jax
import jax.numpy as jnp
from jax import lax
from jax.experimental import pallas as pl
from jax.experimental.pallas import tpu as pltpu
from jax.experimental.pallas import tpu_sc as plsc

T = 2048
D = 1024
E = 8
F = 1536
FS = 4096

TR = 256
NP = 6144
NT = NP // TR
TFS = 512

_NC = 2
_NS = 16
_NW = _NC * _NS


def _routing_body(x_ref, router_ref, dest_ref, gates_ref, te_ref,
                  m_ref, ranks_ref):
    x = x_ref[...]
    scores = jnp.dot(x, router_ref[...], preferred_element_type=jnp.float32)
    lane = lax.broadcasted_iota(jnp.int32, (T, E), 1)
    m1 = jnp.max(scores, axis=1, keepdims=True)
    i1 = jnp.min(jnp.where(scores == m1, lane, E), axis=1, keepdims=True)
    mask1 = lane == i1
    scores2 = jnp.where(mask1, -jnp.inf, scores)
    m2 = jnp.max(scores2, axis=1, keepdims=True)
    i2 = jnp.min(jnp.where(scores2 == m2, lane, E), axis=1, keepdims=True)
    mask2 = lane == i2
    g0 = jax.nn.sigmoid(m1)
    g1 = jax.nn.sigmoid(m2)

    m_ref[...] = (mask1 | mask2).astype(jnp.float32)

    r_i = lax.broadcasted_iota(jnp.int32, (256, 256), 0)
    c_i = lax.broadcasted_iota(jnp.int32, (256, 256), 1)
    tri = (r_i > c_i).astype(jnp.float32)

    def body(c, carry):
        m_c = m_ref[pl.ds(c * 256, 256), :]
        ranks_ref[pl.ds(c * 256, 256), :] = (
            jnp.dot(tri, m_c, preferred_element_type=jnp.float32) + carry)
        return carry + jnp.sum(m_c, axis=0, keepdims=True)

    cnt = lax.fori_loop(0, T // 256, body, jnp.zeros((1, E), jnp.float32))

    aligned = jnp.ceil(cnt * (1.0 / TR)) * TR
    e_r = lax.broadcasted_iota(jnp.int32, (E, E), 0)
    e_c = lax.broadcasted_iota(jnp.int32, (E, E), 1)
    m8 = (e_r < e_c).astype(jnp.float32)
    start = jnp.dot(aligned, m8, preferred_element_type=jnp.float32)

    pos = start + ranks_ref[...]
    d0 = jnp.sum(pos * mask1.astype(jnp.float32), axis=1, keepdims=True)
    d1 = jnp.sum(pos * mask2.astype(jnp.float32), axis=1, keepdims=True)
    dest_ref[...] = jnp.concatenate([d0, d1], axis=1).astype(jnp.int32)
    gates_ref[...] = jnp.concatenate([g0, g1], axis=1)

    jlane = lax.broadcasted_iota(jnp.int32, (1, 32), 1)
    jstart = (jlane * TR).astype(jnp.float32)
    lane8 = lax.broadcasted_iota(jnp.int32, (1, E), 1)
    tot = jnp.sum(aligned, axis=1, keepdims=True)
    te_acc = jnp.zeros((1, 32), jnp.float32)
    for e in range(E):
        s_e = jnp.sum(start * (lane8 == e), axis=1, keepdims=True)
        a_e = jnp.sum(aligned * (lane8 == e), axis=1, keepdims=True)
        te_acc += e * ((jstart >= s_e) & (jstart < s_e + a_e)).astype(jnp.float32)
    te_acc += (E - 1) * (jstart >= tot).astype(jnp.float32)
    te_final = jnp.where(jlane == NT, tot * (1.0 / TR), te_acc)
    te_ref[...] = te_final.astype(jnp.int32)


def _routing(x, router_DE):
    return pl.pallas_call(
        _routing_body,
        grid=(1,),
        in_specs=[
            pl.BlockSpec((T, D), lambda i: (0, 0)),
            pl.BlockSpec((D, E), lambda i: (0, 0)),
        ],
        out_specs=[
            pl.BlockSpec((T, 2), lambda i: (0, 0)),
            pl.BlockSpec((T, 2), lambda i: (0, 0)),
            pl.BlockSpec((1, 32), lambda i: (0, 0)),
        ],
        out_shape=[
            jax.ShapeDtypeStruct((T, 2), jnp.int32),
            jax.ShapeDtypeStruct((T, 2), jnp.float32),
            jax.ShapeDtypeStruct((1, 32), jnp.int32),
        ],
        scratch_shapes=[
            pltpu.VMEM((T, E), jnp.float32),
            pltpu.VMEM((T, E), jnp.float32),
        ],
    )(x, router_DE)


@functools.lru_cache(maxsize=1)
def _get_sc_dispatch():
    @functools.partial(
        pl.kernel,
        mesh=plsc.VectorSubcoreMesh(core_axis_name="c", subcore_axis_name="s",
                                    num_cores=_NC, num_subcores=_NS),
        out_type=[
            jax.ShapeDtypeStruct((NP, D), jnp.float32),
            jax.ShapeDtypeStruct((NP,), jnp.float32),
        ],
        scratch_types=[
            pltpu.VMEM((64,), jnp.int32),
            pltpu.VMEM((64,), jnp.float32),
            pltpu.VMEM((64, D), jnp.float32),
            pltpu.SemaphoreType.DMA,
        ],
    )
    def _sc_dispatch(x_hbm, dest_hbm, gates_hbm, xs_hbm, gs_hbm,
                     idx_v, g_v, x_v, sem):
        wid = lax.axis_index("s") * _NC + lax.axis_index("c")
        k = wid // 16
        tbase = (wid % 16) * 128
        for c in range(2):
            toff = tbase + c * 64
            eoff = k * T + toff
            pltpu.sync_copy(dest_hbm.at[pl.ds(eoff, 64)], idx_v)
            pltpu.sync_copy(gates_hbm.at[pl.ds(eoff, 64)], g_v)
            pltpu.sync_copy(x_hbm.at[pl.ds(toff, 64)], x_v)
            pltpu.async_copy(x_v, xs_hbm.at[idx_v], sem).wait()
            pltpu.async_copy(g_v, gs_hbm.at[idx_v], sem).wait()

    return _sc_dispatch


def _grouped_body(te_ref, xs_ref, gs_ref, w13_ref, w2_ref, ys_ref):
    i = pl.program_id(0)

    @pl.when(i < te_ref[NT])
    def _():
        x = xs_ref[...]
        h = lax.dot_general(x, w13_ref[0], (((1,), (1,)), ((), ())),
                            preferred_element_type=jnp.float32)
        h1 = h[:, :F]
        h3 = h[:, F:]
        act = (h1 * jax.nn.sigmoid(h1)) * h3 * gs_ref[...]
        ys_ref[...] = lax.dot_general(act, w2_ref[0], (((1,), (1,)), ((), ())),
                                      preferred_element_type=jnp.float32)


def _grouped(te25, xs, gs, w13, w2):
    return pl.pallas_call(
        _grouped_body,
        grid_spec=pltpu.PrefetchScalarGridSpec(
            num_scalar_prefetch=1,
            grid=(NT,),
            in_specs=[
                pl.BlockSpec((TR, D), lambda i, te: (i, 0)),
                pl.BlockSpec((TR, 1), lambda i, te: (i, 0)),
                pl.BlockSpec((1, 2 * F, D), lambda i, te: (te[i], 0, 0)),
                pl.BlockSpec((1, D, F), lambda i, te: (te[i], 0, 0)),
            ],
            out_specs=pl.BlockSpec((TR, D), lambda i, te: (i, 0)),
        ),
        out_shape=jax.ShapeDtypeStruct((NP, D), jnp.float32),
        compiler_params=pltpu.CompilerParams(
            dimension_semantics=("arbitrary",)),
    )(te25, xs, gs, w13, w2)


@functools.lru_cache(maxsize=1)
def _get_sc_combine():
    @functools.partial(
        pl.kernel,
        mesh=plsc.VectorSubcoreMesh(core_axis_name="c", subcore_axis_name="s",
                                    num_cores=_NC, num_subcores=_NS),
        out_type=jax.ShapeDtypeStruct((T, D), jnp.float32),
        scratch_types=[
            pltpu.VMEM((32,), jnp.int32),
            pltpu.VMEM((32,), jnp.int32),
            pltpu.VMEM((32, D), jnp.float32),
            pltpu.VMEM((32, D), jnp.float32),
            pltpu.SemaphoreType.DMA,
        ],
    )
    def _sc_combine(ys_hbm, dest_hbm, out_hbm, i0_v, i1_v, y0_v, y1_v, sem):
        wid = lax.axis_index("s") * _NC + lax.axis_index("c")
        tbase = wid * 64
        for c in range(2):
            toff = tbase + c * 32
            pltpu.sync_copy(dest_hbm.at[pl.ds(toff, 32)], i0_v)
            pltpu.sync_copy(dest_hbm.at[pl.ds(T + toff, 32)], i1_v)
            pltpu.async_copy(ys_hbm.at[i0_v], y0_v, sem).wait()
            pltpu.async_copy(ys_hbm.at[i1_v], y1_v, sem).wait()

            def add_row(r, carry):
                for l in range(D // 16):
                    sl = pl.ds(l * 16, 16)
                    y0_v[r, sl] = y0_v[r, sl] + y1_v[r, sl]
                return carry

            lax.fori_loop(0, 32, add_row, 0)
            pltpu.sync_copy(y0_v, out_hbm.at[pl.ds(toff, 32)])

    return _sc_combine


def _shared_body(x_ref, routed_ref, w1s_ref, w3s_ref, w2s_ref, out_ref):
    fs = pl.program_id(1)

    @pl.when(fs == 0)
    def _():
        out_ref[...] = routed_ref[...]

    x = x_ref[...]
    h1 = lax.dot_general(x, w1s_ref[...], (((1,), (1,)), ((), ())),
                         preferred_element_type=jnp.float32)
    h3 = lax.dot_general(x, w3s_ref[...], (((1,), (1,)), ((), ())),
                         preferred_element_type=jnp.float32)
    act = (h1 * jax.nn.sigmoid(h1)) * h3
    out_ref[...] += lax.dot_general(act, w2s_ref[...], (((1,), (1,)), ((), ())),
                                    preferred_element_type=jnp.float32)


def _shared(x, routed, w13_shared, w2_shared):
    nfs = FS // TFS
    return pl.pallas_call(
        _shared_body,
        grid=(1, nfs),
        in_specs=[
            pl.BlockSpec((T, D), lambda t, f: (0, 0)),
            pl.BlockSpec((T, D), lambda t, f: (0, 0)),
            pl.BlockSpec((TFS, D), lambda t, f: (f, 0)),
            pl.BlockSpec((TFS, D), lambda t, f: (nfs + f, 0)),
            pl.BlockSpec((D, TFS), lambda t, f: (0, f)),
        ],
        out_specs=pl.BlockSpec((T, D), lambda t, f: (0, 0)),
        out_shape=jax.ShapeDtypeStruct((T, D), jnp.float32),
        compiler_params=pltpu.CompilerParams(
            dimension_semantics=("arbitrary", "arbitrary")),
    )(x, routed, w13_shared, w13_shared, w2_shared)


@jax.jit
def kernel(x, router_DE, w13, w2, w13_shared, w2_shared):
    dest, gates, te = _routing(x, router_DE)
    dest_flat = jnp.transpose(dest).reshape(2 * T)
    gates_flat = jnp.transpose(gates).reshape(2 * T)
    te25 = te.reshape(32)[:NT + 1]
    xs, gs = _get_sc_dispatch()(x, dest_flat, gates_flat)
    ys = _grouped(te25, xs, gs.reshape(NP, 1), w13, w2)
    routed = _get_sc_combine()(ys, dest_flat)
    return _shared(x, routed, w13_shared, w2_shared)


# speedup vs baseline: 1.1230x; 1.1230x over previous
import functools

import jax
import jax.numpy as jnp
from jax import lax
from jax.experimental import pallas as pl
from jax.experimental.pallas import tpu as pltpu
from jax.experimental.pallas import tpu_sc as plsc

T = 2048
D = 1024
E = 8
F = 1536
FS = 4096

TR = 256
NP = 6144
NT = NP // TR
TFS = 512

_NC = 2
_NS = 16
_NW = _NC * _NS


def _routing_body(x_ref, router_ref, dest_ref, gates_ref, te_ref,
                  m_ref, ranks_ref):
    x = x_ref[...]
    scores = jnp.dot(x, router_ref[...], preferred_element_type=jnp.float32)
    lane = lax.broadcasted_iota(jnp.int32, (T, E), 1)
    m1 = jnp.max(scores, axis=1, keepdims=True)
    i1 = jnp.min(jnp.where(scores == m1, lane, E), axis=1, keepdims=True)
    mask1 = lane == i1
    scores2 = jnp.where(mask1, -jnp.inf, scores)
    m2 = jnp.max(scores2, axis=1, keepdims=True)
    i2 = jnp.min(jnp.where(scores2 == m2, lane, E), axis=1, keepdims=True)
    mask2 = lane == i2
    g0 = jax.nn.sigmoid(m1)
    g1 = jax.nn.sigmoid(m2)

    m_ref[...] = (mask1 | mask2).astype(jnp.float32)

    r_i = lax.broadcasted_iota(jnp.int32, (256, 256), 0)
    c_i = lax.broadcasted_iota(jnp.int32, (256, 256), 1)
    tri = (r_i > c_i).astype(jnp.float32)

    def body(c, carry):
        m_c = m_ref[pl.ds(c * 256, 256), :]
        ranks_ref[pl.ds(c * 256, 256), :] = (
            jnp.dot(tri, m_c, preferred_element_type=jnp.float32) + carry)
        return carry + jnp.sum(m_c, axis=0, keepdims=True)

    cnt = lax.fori_loop(0, T // 256, body, jnp.zeros((1, E), jnp.float32))

    aligned = jnp.ceil(cnt * (1.0 / TR)) * TR
    e_r = lax.broadcasted_iota(jnp.int32, (E, E), 0)
    e_c = lax.broadcasted_iota(jnp.int32, (E, E), 1)
    m8 = (e_r < e_c).astype(jnp.float32)
    start = jnp.dot(aligned, m8, preferred_element_type=jnp.float32)

    pos = start + ranks_ref[...]
    d0 = jnp.sum(pos * mask1.astype(jnp.float32), axis=1, keepdims=True)
    d1 = jnp.sum(pos * mask2.astype(jnp.float32), axis=1, keepdims=True)
    dest_ref[...] = jnp.concatenate([d0, d1], axis=1).astype(jnp.int32)
    gates_ref[...] = jnp.concatenate([g0, g1], axis=1)

    jlane = lax.broadcasted_iota(jnp.int32, (1, 32), 1)
    jstart = (jlane * TR).astype(jnp.float32)
    lane8 = lax.broadcasted_iota(jnp.int32, (1, E), 1)
    tot = jnp.sum(aligned, axis=1, keepdims=True)
    te_acc = jnp.zeros((1, 32), jnp.float32)
    for e in range(E):
        s_e = jnp.sum(start * (lane8 == e), axis=1, keepdims=True)
        a_e = jnp.sum(aligned * (lane8 == e), axis=1, keepdims=True)
        te_acc += e * ((jstart >= s_e) & (jstart < s_e + a_e)).astype(jnp.float32)
    te_acc += (E - 1) * (jstart >= tot).astype(jnp.float32)
    te_final = jnp.where(jlane == NT, tot * (1.0 / TR), te_acc)
    te_ref[...] = te_final.astype(jnp.int32)


def _routing(x, router_DE):
    return pl.pallas_call(
        _routing_body,
        grid=(1,),
        in_specs=[
            pl.BlockSpec((T, D), lambda i: (0, 0)),
            pl.BlockSpec((D, E), lambda i: (0, 0)),
        ],
        out_specs=[
            pl.BlockSpec((T, 2), lambda i: (0, 0)),
            pl.BlockSpec((T, 2), lambda i: (0, 0)),
            pl.BlockSpec((1, 32), lambda i: (0, 0)),
        ],
        out_shape=[
            jax.ShapeDtypeStruct((T, 2), jnp.int32),
            jax.ShapeDtypeStruct((T, 2), jnp.float32),
            jax.ShapeDtypeStruct((1, 32), jnp.int32),
        ],
        scratch_shapes=[
            pltpu.VMEM((T, E), jnp.float32),
            pltpu.VMEM((T, E), jnp.float32),
        ],
    )(x, router_DE)


@functools.lru_cache(maxsize=1)
def _get_sc_dispatch():
    @functools.partial(
        pl.kernel,
        mesh=plsc.VectorSubcoreMesh(core_axis_name="c", subcore_axis_name="s",
                                    num_cores=_NC, num_subcores=_NS),
        out_type=[
            jax.ShapeDtypeStruct((NP, D), jnp.float32),
            jax.ShapeDtypeStruct((NP,), jnp.float32),
        ],
        scratch_types=[
            pltpu.VMEM((64,), jnp.int32),
            pltpu.VMEM((64,), jnp.float32),
            pltpu.VMEM((64, D), jnp.float32),
            pltpu.SemaphoreType.DMA,
        ],
    )
    def _sc_dispatch(x_hbm, dest_hbm, gates_hbm, xs_hbm, gs_hbm,
                     idx_v, g_v, x_v, sem):
        wid = lax.axis_index("s") * _NC + lax.axis_index("c")
        k = wid // 16
        tbase = (wid % 16) * 128
        for c in range(2):
            toff = tbase + c * 64
            eoff = k * T + toff
            pltpu.sync_copy(dest_hbm.at[pl.ds(eoff, 64)], idx_v)
            pltpu.sync_copy(gates_hbm.at[pl.ds(eoff, 64)], g_v)
            pltpu.sync_copy(x_hbm.at[pl.ds(toff, 64)], x_v)
            pltpu.async_copy(x_v, xs_hbm.at[idx_v], sem).wait()
            pltpu.async_copy(g_v, gs_hbm.at[idx_v], sem).wait()

    return _sc_dispatch


def _grouped_body(te_ref, xs_ref, gs_ref, w13_ref, w2_ref, ys_ref):
    i = pl.program_id(0)

    @pl.when(i < te_ref[NT])
    def _():
        x = xs_ref[...]
        h = lax.dot_general(x, w13_ref[0], (((1,), (1,)), ((), ())),
                            preferred_element_type=jnp.float32)
        h1 = h[:, :F]
        h3 = h[:, F:]
        act = (h1 * jax.nn.sigmoid(h1)) * h3 * gs_ref[...]
        ys_ref[...] = lax.dot_general(act, w2_ref[0], (((1,), (1,)), ((), ())),
                                      preferred_element_type=jnp.float32)


def _grouped(te25, xs, gs, w13, w2):
    return pl.pallas_call(
        _grouped_body,
        grid_spec=pltpu.PrefetchScalarGridSpec(
            num_scalar_prefetch=1,
            grid=(NT,),
            in_specs=[
                pl.BlockSpec((TR, D), lambda i, te: (i, 0)),
                pl.BlockSpec((TR, 1), lambda i, te: (i, 0)),
                pl.BlockSpec((1, 2 * F, D), lambda i, te: (te[i], 0, 0)),
                pl.BlockSpec((1, D, F), lambda i, te: (te[i], 0, 0)),
            ],
            out_specs=pl.BlockSpec((TR, D), lambda i, te: (i, 0)),
        ),
        out_shape=jax.ShapeDtypeStruct((NP, D), jnp.float32),
        compiler_params=pltpu.CompilerParams(
            dimension_semantics=("arbitrary",)),
    )(te25, xs, gs, w13, w2)


@functools.lru_cache(maxsize=1)
def _get_sc_combine():
    @functools.partial(
        pl.kernel,
        mesh=plsc.VectorSubcoreMesh(core_axis_name="c", subcore_axis_name="s",
                                    num_cores=_NC, num_subcores=_NS),
        out_type=jax.ShapeDtypeStruct((T, D), jnp.float32),
        scratch_types=[
            pltpu.VMEM((32,), jnp.int32),
            pltpu.VMEM((32,), jnp.int32),
            pltpu.VMEM((32, D), jnp.float32),
            pltpu.VMEM((32, D), jnp.float32),
            pltpu.VMEM((32, D), jnp.float32),
            pltpu.SemaphoreType.DMA,
        ],
    )
    def _sc_combine(ys_hbm, dest_hbm, shared_hbm, out_hbm,
                    i0_v, i1_v, y0_v, y1_v, s_v, sem):
        wid = lax.axis_index("s") * _NC + lax.axis_index("c")
        tbase = wid * 64
        for c in range(2):
            toff = tbase + c * 32
            pltpu.sync_copy(dest_hbm.at[pl.ds(toff, 32)], i0_v)
            pltpu.sync_copy(dest_hbm.at[pl.ds(T + toff, 32)], i1_v)
            h0 = pltpu.async_copy(ys_hbm.at[i0_v], y0_v, sem)
            h1 = pltpu.async_copy(ys_hbm.at[i1_v], y1_v, sem)
            pltpu.sync_copy(shared_hbm.at[pl.ds(toff, 32)], s_v)
            h0.wait()
            h1.wait()

            def add_row(r, carry):
                for l in range(D // 16):
                    sl = pl.ds(l * 16, 16)
                    y0_v[r, sl] = y0_v[r, sl] + y1_v[r, sl] + s_v[r, sl]
                return carry

            lax.fori_loop(0, 32, add_row, 0)
            pltpu.sync_copy(y0_v, out_hbm.at[pl.ds(toff, 32)])

    return _sc_combine


def _shared_body(x_ref, w1s_ref, w3s_ref, w2s_ref, out_ref):
    fs = pl.program_id(1)

    x = x_ref[...]
    h1 = lax.dot_general(x, w1s_ref[...], (((1,), (1,)), ((), ())),
                         preferred_element_type=jnp.float32)
    h3 = lax.dot_general(x, w3s_ref[...], (((1,), (1,)), ((), ())),
                         preferred_element_type=jnp.float32)
    act = (h1 * jax.nn.sigmoid(h1)) * h3
    contrib = lax.dot_general(act, w2s_ref[...], (((1,), (1,)), ((), ())),
                              preferred_element_type=jnp.float32)

    @pl.when(fs == 0)
    def _():
        out_ref[...] = contrib

    @pl.when(fs != 0)
    def _():
        out_ref[...] += contrib


def _shared(x, w13_shared, w2_shared):
    nfs = FS // TFS
    return pl.pallas_call(
        _shared_body,
        grid=(1, nfs),
        in_specs=[
            pl.BlockSpec((T, D), lambda t, f: (0, 0)),
            pl.BlockSpec((TFS, D), lambda t, f: (f, 0)),
            pl.BlockSpec((TFS, D), lambda t, f: (nfs + f, 0)),
            pl.BlockSpec((D, TFS), lambda t, f: (0, f)),
        ],
        out_specs=pl.BlockSpec((T, D), lambda t, f: (0, 0)),
        out_shape=jax.ShapeDtypeStruct((T, D), jnp.float32),
        compiler_params=pltpu.CompilerParams(
            dimension_semantics=("arbitrary", "arbitrary")),
    )(x, w13_shared, w13_shared, w2_shared)


@jax.jit
def kernel(x, router_DE, w13, w2, w13_shared, w2_shared):
    dest, gates, te = _routing(x, router_DE)
    dest_flat = jnp.transpose(dest).reshape(2 * T)
    gates_flat = jnp.transpose(gates).reshape(2 * T)
    te25 = te.reshape(32)[:NT + 1]
    shared = _shared(x, w13_shared, w2_shared)
    xs, gs = _get_sc_dispatch()(x, dest_flat, gates_flat)
    ys = _grouped(te25, xs, gs.reshape(NP, 1), w13, w2)
    return _get_sc_combine()(ys, dest_flat, shared)
